# baseline (speedup 1.0000x reference)
import functools

import jax
import jax.numpy as jnp
from jax.experimental import pallas as pl
from jax.experimental.pallas import tpu as pltpu

SEQ = 8192
D = 768
H = 24576
K = 150

BT = 256
BH = 2048


def _enc_body(x_ref, w_ref, b_ref, out_ref):
    out_ref[...] = jax.lax.dot_general(
        x_ref[...], w_ref[...], (((1,), (1,)), ((), ())),
        preferred_element_type=jnp.float32) + b_ref[...][None, :]


def _encoder(x, W_enc, b_enc):
    grid = (H // BH, SEQ // BT)
    return pl.pallas_call(
        _enc_body,
        grid=grid,
        in_specs=[
            pl.BlockSpec((BT, D), lambda h, t: (t, 0)),
            pl.BlockSpec((BH, D), lambda h, t: (h, 0)),
            pl.BlockSpec((BH,), lambda h, t: (h,)),
        ],
        out_specs=pl.BlockSpec((BT, BH), lambda h, t: (t, h)),
        out_shape=jax.ShapeDtypeStruct((SEQ, H), jnp.float32),
    )(x, W_enc, b_enc)


BTS = 64
CM = 64
SEL_A = 22
SEL_B = 18


def _sel_body(pre_ref, t_ref):
    x = pre_ref[...]
    cm = jnp.max(x.reshape(BTS, H // CM, CM), axis=2)
    rowmax = jnp.max(cm, axis=1)
    lo0 = jnp.min(cm, axis=1) - 1.0

    def it_cm(_, c):
        lo, hi = c
        mid = 0.5 * (lo + hi)
        cnt = jnp.sum(jnp.where(cm > mid[:, None], 1.0, 0.0), axis=1)
        pred = cnt >= K
        return (jnp.where(pred, mid, lo), jnp.where(pred, hi, mid))

    loA, _ = jax.lax.fori_loop(0, SEL_A, it_cm, (lo0, rowmax))

    def it_x(_, c):
        lo, hi = c
        mid = 0.5 * (lo + hi)
        cnt = jnp.sum(jnp.where(x > mid[:, None], 1.0, 0.0), axis=1)
        pred = cnt >= K
        return (jnp.where(pred, mid, lo), jnp.where(pred, hi, mid))

    lo, _ = jax.lax.fori_loop(0, SEL_B, it_x, (loA, rowmax))
    t_ref[...] = lo[None, None, :]


def _select_threshold(pre):
    out = pl.pallas_call(
        _sel_body,
        grid=(SEQ // BTS,),
        in_specs=[pl.BlockSpec((BTS, H), lambda t: (t, 0))],
        out_specs=pl.BlockSpec((1, 1, BTS), lambda t: (t, 0, 0)),
        out_shape=jax.ShapeDtypeStruct((SEQ // BTS, 1, BTS), jnp.float32),
    )(pre)
    return out.reshape(SEQ)


BTD = 512


def _dec_body(p_ref, t_ref, w_ref, b_ref, out_ref):
    k = pl.program_id(1)

    @pl.when(k == 0)
    def _init():
        out_ref[...] = jnp.broadcast_to(b_ref[...][None, :], out_ref.shape)

    p = p_ref[...]
    s = jnp.where(p > t_ref[...][:, None], p, 0.0).astype(jnp.bfloat16)
    out_ref[...] += jax.lax.dot_general(
        s, w_ref[...], (((1,), (1,)), ((), ())),
        preferred_element_type=jnp.float32)


def _decoder(pre, thr, W_dec_bf16, b_dec):
    grid = (SEQ // BTD, H // BH)
    return pl.pallas_call(
        _dec_body,
        grid=grid,
        in_specs=[
            pl.BlockSpec((BTD, BH), lambda t, k: (t, k)),
            pl.BlockSpec((BTD,), lambda t, k: (t,)),
            pl.BlockSpec((D, BH), lambda t, k: (0, k)),
            pl.BlockSpec((D,), lambda t, k: (0,)),
        ],
        out_specs=pl.BlockSpec((BTD, D), lambda t, k: (t, 0)),
        out_shape=jax.ShapeDtypeStruct((SEQ, D), jnp.float32),
    )(pre, thr, W_dec_bf16, b_dec)


def kernel(llm_activations, W_enc, b_enc, W_dec, b_dec):
    x = llm_activations.reshape(SEQ, D)
    pre = _encoder(x, W_enc, b_enc)
    thr = _select_threshold(pre)
    out = _decoder(pre, thr, W_dec.astype(jnp.bfloat16), b_dec)
    return out.reshape(1, SEQ, D)


# speedup vs baseline: 3.4510x; 3.4510x over previous
import functools

import jax
import jax.numpy as jnp
from jax.experimental import pallas as pl
from jax.experimental.pallas import tpu as pltpu

SEQ = 8192
D = 768
H = 24576
K = 150

BT = 256
BH = 2048


def _enc_body(x_ref, w_ref, b_ref, out_ref):
    out_ref[...] = jax.lax.dot_general(
        x_ref[...], w_ref[...], (((1,), (1,)), ((), ())),
        preferred_element_type=jnp.float32) + b_ref[...][None, :]


def _encoder(x, W_enc, b_enc):
    grid = (H // BH, SEQ // BT)
    return pl.pallas_call(
        _enc_body,
        grid=grid,
        in_specs=[
            pl.BlockSpec((BT, D), lambda h, t: (t, 0)),
            pl.BlockSpec((BH, D), lambda h, t: (h, 0)),
            pl.BlockSpec((BH,), lambda h, t: (h,)),
        ],
        out_specs=pl.BlockSpec((BT, BH), lambda h, t: (t, h)),
        out_shape=jax.ShapeDtypeStruct((SEQ, H), jnp.float32),
    )(x, W_enc, b_enc)


BTS = 128
SEL_ITERS = 21


def _sel_body(pre_ref, t_ref):
    x = pre_ref[...]
    lo0 = jnp.min(x, axis=1) - 1.0
    hi0 = jnp.max(x, axis=1)

    def it_x(_, c):
        lo, hi = c
        mid = 0.5 * (lo + hi)
        cnt = jnp.sum(jnp.where(x > mid[:, None], 1.0, 0.0), axis=1)
        pred = cnt >= K
        return (jnp.where(pred, mid, lo), jnp.where(pred, hi, mid))

    lo, _ = jax.lax.fori_loop(0, SEL_ITERS, it_x, (lo0, hi0))
    t_ref[...] = lo[None, None, :]


def _select_threshold(pre):
    out = pl.pallas_call(
        _sel_body,
        grid=(SEQ // BTS,),
        in_specs=[pl.BlockSpec((BTS, H), lambda t: (t, 0))],
        out_specs=pl.BlockSpec((1, 1, BTS), lambda t: (t, 0, 0)),
        out_shape=jax.ShapeDtypeStruct((SEQ // BTS, 1, BTS), jnp.float32),
    )(pre)
    return out.reshape(SEQ)


BTD = 512


def _dec_body(p_ref, t_ref, w_ref, b_ref, out_ref):
    k = pl.program_id(1)

    @pl.when(k == 0)
    def _init():
        out_ref[...] = jnp.broadcast_to(b_ref[...][None, :], out_ref.shape)

    p = p_ref[...]
    s = jnp.where(p > t_ref[...][:, None], p, 0.0).astype(jnp.bfloat16)
    out_ref[...] += jax.lax.dot_general(
        s, w_ref[...], (((1,), (1,)), ((), ())),
        preferred_element_type=jnp.float32)


def _decoder(pre, thr, W_dec_bf16, b_dec):
    grid = (SEQ // BTD, H // BH)
    return pl.pallas_call(
        _dec_body,
        grid=grid,
        in_specs=[
            pl.BlockSpec((BTD, BH), lambda t, k: (t, k)),
            pl.BlockSpec((BTD,), lambda t, k: (t,)),
            pl.BlockSpec((D, BH), lambda t, k: (0, k)),
            pl.BlockSpec((D,), lambda t, k: (0,)),
        ],
        out_specs=pl.BlockSpec((BTD, D), lambda t, k: (t, 0)),
        out_shape=jax.ShapeDtypeStruct((SEQ, D), jnp.float32),
    )(pre, thr, W_dec_bf16, b_dec)


def kernel(llm_activations, W_enc, b_enc, W_dec, b_dec):
    x = llm_activations.reshape(SEQ, D)
    pre = _encoder(x, W_enc, b_enc)
    thr = _select_threshold(pre)
    out = _decoder(pre, thr, W_dec.astype(jnp.bfloat16), b_dec)
    return out.reshape(1, SEQ, D)
